# Initial kernel scaffold; baseline (speedup 1.0000x reference)
import jax
import jax.numpy as jnp
from jax.experimental import pallas as pl


def kernel(input_ids, token_type_ids, visual_embeddings, visual_embeddings_type, word_emb, pos_emb, tok_type_emb, tok_type_vis_emb, proj_W, proj_b, ln_gamma, ln_beta):
    raise NotImplementedError("write your pallas kernel here")


# speedup vs baseline: 1.7708x; 1.7708x over previous
import functools

import jax
import jax.numpy as jnp
from jax import lax
from jax.experimental import pallas as pl
from jax.experimental.pallas import tpu as pltpu
from jax.experimental.pallas import tpu_sc as plsc

VOCAB = 30522
HIDDEN = 768
MAX_POS = 512
B, S, L = 64, 512, 100
VIS_DIM = 2048
EPS = 1e-12


_NC, _NS = 2, 16
_NW = _NC * _NS
_BS = B * S
_PER_W = _BS // _NW
_CHUNK = 64
_NCHUNK = _PER_W // _CHUNK


def _sc_gather_body(table_hbm, idx_hbm, out_hbm, idx_v, buf_v,
                    gsem0, gsem1, ssem0, ssem1):
    wid = lax.axis_index("s") * _NC + lax.axis_index("c")
    base = wid * _PER_W
    pltpu.sync_copy(idx_hbm.at[pl.ds(base, _PER_W)], idx_v)

    gsems = (gsem0, gsem1)
    ssems = (ssem0, ssem1)

    def start_gather(i):
        bslot = i % 2
        return pltpu.async_copy(
            table_hbm.at[idx_v.at[pl.ds(i * _CHUNK, _CHUNK)]],
            buf_v.at[bslot], gsems[bslot])

    def start_scatter(i):
        bslot = i % 2
        return pltpu.async_copy(
            buf_v.at[bslot],
            out_hbm.at[pl.ds(base + i * _CHUNK, _CHUNK)], ssems[bslot])

    gathers = [None] * _NCHUNK
    scatters = [None] * _NCHUNK
    gathers[0] = start_gather(0)
    for i in range(_NCHUNK):
        if i + 1 < _NCHUNK:
            if i >= 1:
                scatters[i - 1].wait()
            gathers[i + 1] = start_gather(i + 1)
        gathers[i].wait()
        scatters[i] = start_scatter(i)
    scatters[_NCHUNK - 2].wait()
    scatters[_NCHUNK - 1].wait()


@functools.cache
def _sc_gather_kernel():
    return pl.kernel(
        _sc_gather_body,
        out_type=jax.ShapeDtypeStruct((_BS, HIDDEN), jnp.float32),
        mesh=plsc.VectorSubcoreMesh(core_axis_name="c", subcore_axis_name="s"),
        scratch_types=[
            pltpu.VMEM((_PER_W,), jnp.int32),
            pltpu.VMEM((2, _CHUNK, HIDDEN), jnp.float32),
            pltpu.SemaphoreType.DMA,
            pltpu.SemaphoreType.DMA,
            pltpu.SemaphoreType.DMA,
            pltpu.SemaphoreType.DMA,
        ],
    )


def _layer_norm(x, gamma, beta):
    mu = jnp.mean(x, axis=-1, keepdims=True)
    xc = x - mu
    var = jnp.mean(xc * xc, axis=-1, keepdims=True)
    return xc * lax.rsqrt(var + EPS) * gamma + beta


def _tc_body(gw_ref, pos_ref, tt_ref, tte_ref, vis_ref, vt_ref, tve_ref,
             w_ref, b_ref, gamma_ref, beta_ref, out_ref):
    gamma = gamma_ref[...]
    beta = beta_ref[...]

    tt = tt_ref[0, 0].astype(jnp.float32)[:, None]
    tte0 = tte_ref[0][None, :]
    tte_d = (tte_ref[1] - tte_ref[0])[None, :]
    e = gw_ref[0] + pos_ref[...] + tte0 + tt * tte_d
    out_ref[0, :S, :] = _layer_norm(e, gamma, beta)

    proj = jnp.dot(vis_ref[0], w_ref[...],
                   preferred_element_type=jnp.float32)
    vt = vt_ref[0, 0].astype(jnp.float32)[:, None]
    tve0 = tve_ref[0][None, :]
    tve_d = (tve_ref[1] - tve_ref[0])[None, :]
    v = proj + b_ref[...] + tve0 + vt * tve_d
    out_ref[0, S:S + L, :] = _layer_norm(v, gamma, beta)


def _tc_fused(gw, pos_emb, tt3, tok_type_emb, vis, vt3, tok_type_vis_emb,
              proj_W, proj_b2, gamma2, beta2):
    return pl.pallas_call(
        _tc_body,
        grid=(B,),
        in_specs=[
            pl.BlockSpec((1, S, HIDDEN), lambda b: (b, 0, 0)),
            pl.BlockSpec((MAX_POS, HIDDEN), lambda b: (0, 0)),
            pl.BlockSpec((1, 1, S), lambda b: (b, 0, 0)),
            pl.BlockSpec((2, HIDDEN), lambda b: (0, 0)),
            pl.BlockSpec((1, L, VIS_DIM), lambda b: (b, 0, 0)),
            pl.BlockSpec((1, 1, L), lambda b: (b, 0, 0)),
            pl.BlockSpec((2, HIDDEN), lambda b: (0, 0)),
            pl.BlockSpec((VIS_DIM, HIDDEN), lambda b: (0, 0)),
            pl.BlockSpec((1, HIDDEN), lambda b: (0, 0)),
            pl.BlockSpec((1, HIDDEN), lambda b: (0, 0)),
            pl.BlockSpec((1, HIDDEN), lambda b: (0, 0)),
        ],
        out_specs=pl.BlockSpec((1, S + L, HIDDEN), lambda b: (b, 0, 0)),
        out_shape=jax.ShapeDtypeStruct((B, S + L, HIDDEN), jnp.float32),
    )(gw, pos_emb, tt3, tok_type_emb, vis, vt3, tok_type_vis_emb,
      proj_W, proj_b2, gamma2, beta2)


def kernel(input_ids, token_type_ids, visual_embeddings, visual_embeddings_type,
           word_emb, pos_emb, tok_type_emb, tok_type_vis_emb,
           proj_W, proj_b, ln_gamma, ln_beta):
    idx = input_ids.reshape(-1).astype(jnp.int32)
    gw = _sc_gather_kernel()(word_emb, idx).reshape(B, S, HIDDEN)
    tt3 = token_type_ids.astype(jnp.int32).reshape(B, 1, S)
    vt3 = visual_embeddings_type.astype(jnp.int32).reshape(B, 1, L)
    return _tc_fused(gw, pos_emb, tt3, tok_type_emb,
                     visual_embeddings, vt3, tok_type_vis_emb,
                     proj_W, proj_b.reshape(1, HIDDEN),
                     ln_gamma.reshape(1, HIDDEN), ln_beta.reshape(1, HIDDEN))


# speedup vs baseline: 1.7846x; 1.0078x over previous
import functools

import jax
import jax.numpy as jnp
from jax import lax
from jax.experimental import pallas as pl
from jax.experimental.pallas import tpu as pltpu
from jax.experimental.pallas import tpu_sc as plsc

VOCAB = 30522
HIDDEN = 768
MAX_POS = 512
B, S, L = 64, 512, 100
VIS_DIM = 2048
EPS = 1e-12


_NC, _NS = 2, 16
_NW = _NC * _NS
_BS = B * S
_PER_W = _BS // _NW
_CHUNK = 64
_NCHUNK = _PER_W // _CHUNK


def _sc_gather_body(table_hbm, idx_hbm, out_hbm, idx_v, buf_v,
                    gsem0, gsem1, ssem0, ssem1):
    wid = lax.axis_index("s") * _NC + lax.axis_index("c")
    base = wid * _PER_W
    pltpu.sync_copy(idx_hbm.at[pl.ds(base, _PER_W)], idx_v)

    gsems = (gsem0, gsem1)
    ssems = (ssem0, ssem1)

    def start_gather(i):
        bslot = i % 2
        return pltpu.async_copy(
            table_hbm.at[idx_v.at[pl.ds(i * _CHUNK, _CHUNK)]],
            buf_v.at[bslot], gsems[bslot])

    def start_scatter(i):
        bslot = i % 2
        return pltpu.async_copy(
            buf_v.at[bslot],
            out_hbm.at[pl.ds(base + i * _CHUNK, _CHUNK)], ssems[bslot])

    gathers = [None] * _NCHUNK
    scatters = [None] * _NCHUNK
    gathers[0] = start_gather(0)
    for i in range(_NCHUNK):
        if i + 1 < _NCHUNK:
            if i >= 1:
                scatters[i - 1].wait()
            gathers[i + 1] = start_gather(i + 1)
        gathers[i].wait()
        scatters[i] = start_scatter(i)
    scatters[_NCHUNK - 2].wait()
    scatters[_NCHUNK - 1].wait()


@functools.cache
def _sc_gather_kernel():
    return pl.kernel(
        _sc_gather_body,
        out_type=jax.ShapeDtypeStruct((_BS, HIDDEN), jnp.float32),
        mesh=plsc.VectorSubcoreMesh(core_axis_name="c", subcore_axis_name="s"),
        scratch_types=[
            pltpu.VMEM((_PER_W,), jnp.int32),
            pltpu.VMEM((2, _CHUNK, HIDDEN), jnp.float32),
            pltpu.SemaphoreType.DMA,
            pltpu.SemaphoreType.DMA,
            pltpu.SemaphoreType.DMA,
            pltpu.SemaphoreType.DMA,
        ],
    )


def _layer_norm(x, gamma, beta):
    mu = jnp.mean(x, axis=-1, keepdims=True)
    xc = x - mu
    var = jnp.mean(xc * xc, axis=-1, keepdims=True)
    return xc * lax.rsqrt(var + EPS) * gamma + beta


def _tc_body(gw_ref, pos_ref, tt_ref, tte_ref, vis_ref, vt_ref, tve_ref,
             w_ref, b_ref, gamma_ref, beta_ref, out_ref):
    gamma = gamma_ref[...]
    beta = beta_ref[...]

    tt = tt_ref[0, 0].astype(jnp.float32)[:, None]
    tte0 = tte_ref[0][None, :]
    tte_d = (tte_ref[1] - tte_ref[0])[None, :]
    e = gw_ref[0] + pos_ref[...] + tte0 + tt * tte_d
    out_ref[0, :S, :] = _layer_norm(e, gamma, beta)

    proj = jnp.dot(vis_ref[0].astype(jnp.bfloat16), w_ref[...],
                   preferred_element_type=jnp.float32)
    vt = vt_ref[0, 0].astype(jnp.float32)[:, None]
    tve0 = tve_ref[0][None, :]
    tve_d = (tve_ref[1] - tve_ref[0])[None, :]
    v = proj + b_ref[...] + tve0 + vt * tve_d
    out_ref[0, S:S + L, :] = _layer_norm(v, gamma, beta)


def _tc_fused(gw, pos_emb, tt3, tok_type_emb, vis, vt3, tok_type_vis_emb,
              proj_W, proj_b2, gamma2, beta2):
    return pl.pallas_call(
        _tc_body,
        grid=(B,),
        in_specs=[
            pl.BlockSpec((1, S, HIDDEN), lambda b: (b, 0, 0)),
            pl.BlockSpec((MAX_POS, HIDDEN), lambda b: (0, 0)),
            pl.BlockSpec((1, 1, S), lambda b: (b, 0, 0)),
            pl.BlockSpec((2, HIDDEN), lambda b: (0, 0)),
            pl.BlockSpec((1, L, VIS_DIM), lambda b: (b, 0, 0)),
            pl.BlockSpec((1, 1, L), lambda b: (b, 0, 0)),
            pl.BlockSpec((2, HIDDEN), lambda b: (0, 0)),
            pl.BlockSpec((VIS_DIM, HIDDEN), lambda b: (0, 0)),
            pl.BlockSpec((1, HIDDEN), lambda b: (0, 0)),
            pl.BlockSpec((1, HIDDEN), lambda b: (0, 0)),
            pl.BlockSpec((1, HIDDEN), lambda b: (0, 0)),
        ],
        out_specs=pl.BlockSpec((1, S + L, HIDDEN), lambda b: (b, 0, 0)),
        out_shape=jax.ShapeDtypeStruct((B, S + L, HIDDEN), jnp.float32),
    )(gw, pos_emb, tt3, tok_type_emb, vis, vt3, tok_type_vis_emb,
      proj_W, proj_b2, gamma2, beta2)


def kernel(input_ids, token_type_ids, visual_embeddings, visual_embeddings_type,
           word_emb, pos_emb, tok_type_emb, tok_type_vis_emb,
           proj_W, proj_b, ln_gamma, ln_beta):
    idx = input_ids.reshape(-1).astype(jnp.int32)
    gw = _sc_gather_kernel()(word_emb, idx).reshape(B, S, HIDDEN)
    tt3 = token_type_ids.astype(jnp.int32).reshape(B, 1, S)
    vt3 = visual_embeddings_type.astype(jnp.int32).reshape(B, 1, L)
    return _tc_fused(gw, pos_emb, tt3, tok_type_emb,
                     visual_embeddings, vt3, tok_type_vis_emb,
                     proj_W.astype(jnp.bfloat16), proj_b.reshape(1, HIDDEN),
                     ln_gamma.reshape(1, HIDDEN), ln_beta.reshape(1, HIDDEN))


# speedup vs baseline: 2.2419x; 1.2562x over previous
import functools

import jax
import jax.numpy as jnp
from jax import lax
from jax.experimental import pallas as pl
from jax.experimental.pallas import tpu as pltpu
from jax.experimental.pallas import tpu_sc as plsc

VOCAB = 30522
HIDDEN = 768
B, S, L = 64, 512, 100
VIS_DIM = 2048
EPS = 1e-12


_NC, _NS = 2, 16
_NW = _NC * _NS
_BS = B * S
_PER_W = _BS // _NW
_CHUNK = 64
_NCHUNK = _PER_W // _CHUNK


def _sc_gather_body(table_hbm, idx_hbm, out_hbm, idx_v, buf_v,
                    gsem0, gsem1, ssem0, ssem1):
    wid = lax.axis_index("s") * _NC + lax.axis_index("c")
    base = wid * _PER_W
    pltpu.sync_copy(idx_hbm.at[pl.ds(base, _PER_W)], idx_v)

    gsems = (gsem0, gsem1)
    ssems = (ssem0, ssem1)

    def start_gather(i):
        bslot = i % 2
        return pltpu.async_copy(
            table_hbm.at[idx_v.at[pl.ds(i * _CHUNK, _CHUNK)]],
            buf_v.at[bslot], gsems[bslot])

    def start_scatter(i):
        bslot = i % 2
        return pltpu.async_copy(
            buf_v.at[bslot],
            out_hbm.at[pl.ds(base + i * _CHUNK, _CHUNK)], ssems[bslot])

    gathers = [None] * _NCHUNK
    scatters = [None] * _NCHUNK
    gathers[0] = start_gather(0)
    for i in range(_NCHUNK):
        if i + 1 < _NCHUNK:
            if i >= 1:
                scatters[i - 1].wait()
            gathers[i + 1] = start_gather(i + 1)
        gathers[i].wait()
        scatters[i] = start_scatter(i)
    scatters[_NCHUNK - 2].wait()
    scatters[_NCHUNK - 1].wait()


@functools.cache
def _sc_gather_kernel():
    return pl.kernel(
        _sc_gather_body,
        out_type=jax.ShapeDtypeStruct((_BS, HIDDEN), jnp.float32),
        mesh=plsc.VectorSubcoreMesh(core_axis_name="c", subcore_axis_name="s"),
        scratch_types=[
            pltpu.VMEM((_PER_W,), jnp.int32),
            pltpu.VMEM((2, _CHUNK, HIDDEN), jnp.float32),
            pltpu.SemaphoreType.DMA,
            pltpu.SemaphoreType.DMA,
            pltpu.SemaphoreType.DMA,
            pltpu.SemaphoreType.DMA,
        ],
    )


_CH = 4
_NT = S // _CH
_NV = L // _CH


def _layer_norm3(x, gamma, beta):
    mu = jnp.mean(x, axis=-1, keepdims=True)
    xc = x - mu
    var = jnp.mean(xc * xc, axis=-1, keepdims=True)
    return xc * lax.rsqrt(var + EPS) * gamma + beta


def _tc_body(gw_ref, pos_ref, tt_ref, tte_ref, vis_ref, vt_ref, tve_ref,
             w_ref, b_ref, gamma_ref, beta_ref, out_ref):
    g = pl.program_id(0)
    gamma = gamma_ref[...][None]
    beta = beta_ref[...][None]

    @pl.when(g < _NT)
    def _text():
        t = tt_ref[0].astype(jnp.float32)[:, :, None]
        tte0 = tte_ref[0][None, None, :]
        tte_d = (tte_ref[1] - tte_ref[0])[None, None, :]
        e = gw_ref[...] + pos_ref[0][:, None, :] + tte0 + t * tte_d
        out_ref[...] = _layer_norm3(e, gamma, beta)

    @pl.when(g >= _NT)
    def _visual():
        v = vis_ref[...].astype(jnp.bfloat16).reshape(_CH * B, VIS_DIM)
        proj = jnp.dot(v, w_ref[...], preferred_element_type=jnp.float32)
        proj = proj.reshape(_CH, B, HIDDEN)
        t = vt_ref[0].astype(jnp.float32)[:, :, None]
        tve0 = tve_ref[0][None, None, :]
        tve_d = (tve_ref[1] - tve_ref[0])[None, None, :]
        ve = proj + b_ref[...][None] + tve0 + t * tve_d
        out_ref[...] = _layer_norm3(ve, gamma, beta)


def _tc_fused(gw_t, pos_emb, tt3, tok_type_emb, vis_t, vt3, tok_type_vis_emb,
              proj_Wb, proj_b2, gamma2, beta2):
    return pl.pallas_call(
        _tc_body,
        grid=(_NT + _NV,),
        in_specs=[
            pl.BlockSpec((_CH, B, HIDDEN),
                         lambda g: (jnp.minimum(g, _NT - 1), 0, 0)),
            pl.BlockSpec((1, _CH, HIDDEN),
                         lambda g: (jnp.minimum(g, _NT - 1), 0, 0)),
            pl.BlockSpec((1, _CH, B),
                         lambda g: (jnp.minimum(g, _NT - 1), 0, 0)),
            pl.BlockSpec((2, HIDDEN), lambda g: (0, 0)),
            pl.BlockSpec((_CH, B, VIS_DIM),
                         lambda g: (jnp.maximum(g - _NT, 0), 0, 0)),
            pl.BlockSpec((1, _CH, B),
                         lambda g: (jnp.maximum(g - _NT, 0), 0, 0)),
            pl.BlockSpec((2, HIDDEN), lambda g: (0, 0)),
            pl.BlockSpec((VIS_DIM, HIDDEN), lambda g: (0, 0)),
            pl.BlockSpec((1, HIDDEN), lambda g: (0, 0)),
            pl.BlockSpec((1, HIDDEN), lambda g: (0, 0)),
            pl.BlockSpec((1, HIDDEN), lambda g: (0, 0)),
        ],
        out_specs=pl.BlockSpec((_CH, B, HIDDEN), lambda g: (g, 0, 0)),
        out_shape=jax.ShapeDtypeStruct((S + L, B, HIDDEN), jnp.float32),
    )(gw_t, pos_emb, tt3, tok_type_emb, vis_t, vt3, tok_type_vis_emb,
      proj_Wb, proj_b2, gamma2, beta2)


def kernel(input_ids, token_type_ids, visual_embeddings, visual_embeddings_type,
           word_emb, pos_emb, tok_type_emb, tok_type_vis_emb,
           proj_W, proj_b, ln_gamma, ln_beta):
    idx_t = input_ids.astype(jnp.int32).T.reshape(-1)
    gw_t = _sc_gather_kernel()(word_emb, idx_t).reshape(S, B, HIDDEN)
    tt3 = token_type_ids.astype(jnp.int32).T.reshape(_NT, _CH, B)
    vt3 = visual_embeddings_type.astype(jnp.int32).T.reshape(_NV, _CH, B)
    vis_t = jnp.transpose(visual_embeddings, (1, 0, 2))
    out_t = _tc_fused(gw_t, pos_emb.reshape(_NT, _CH, HIDDEN), tt3, tok_type_emb,
                      vis_t, vt3, tok_type_vis_emb,
                      proj_W.astype(jnp.bfloat16), proj_b.reshape(1, HIDDEN),
                      ln_gamma.reshape(1, HIDDEN), ln_beta.reshape(1, HIDDEN))
    return jnp.transpose(out_t, (1, 0, 2))


# speedup vs baseline: 3.0705x; 1.3696x over previous
import functools

import jax
import jax.numpy as jnp
from jax import lax
from jax.experimental import pallas as pl
from jax.experimental.pallas import tpu as pltpu
from jax.experimental.pallas import tpu_sc as plsc

VOCAB = 30522
HIDDEN = 768
B, S, L = 64, 512, 100
VIS_DIM = 2048
EPS = 1e-12


_NC, _NS = 2, 16
_NW = _NC * _NS
_BS = B * S
_PER_W = _BS // _NW
_CHUNK = 64
_NCHUNK = _PER_W // _CHUNK


def _sc_gather_body(table_hbm, idx_hbm, out_hbm, idx_v, buf_v,
                    gsem0, gsem1, ssem0, ssem1):
    wid = lax.axis_index("s") * _NC + lax.axis_index("c")
    base = wid * _PER_W
    pltpu.sync_copy(idx_hbm.at[pl.ds(base, _PER_W)], idx_v)

    gsems = (gsem0, gsem1)
    ssems = (ssem0, ssem1)

    def start_gather(i):
        bslot = i % 2
        return pltpu.async_copy(
            table_hbm.at[idx_v.at[pl.ds(i * _CHUNK, _CHUNK)]],
            buf_v.at[bslot], gsems[bslot])

    def start_scatter(i):
        bslot = i % 2
        return pltpu.async_copy(
            buf_v.at[bslot],
            out_hbm.at[pl.ds(base + i * _CHUNK, _CHUNK)], ssems[bslot])

    gathers = [None] * _NCHUNK
    scatters = [None] * _NCHUNK
    gathers[0] = start_gather(0)
    for i in range(_NCHUNK):
        if i + 1 < _NCHUNK:
            if i >= 1:
                scatters[i - 1].wait()
            gathers[i + 1] = start_gather(i + 1)
        gathers[i].wait()
        scatters[i] = start_scatter(i)
    scatters[_NCHUNK - 2].wait()
    scatters[_NCHUNK - 1].wait()


@functools.cache
def _sc_gather_kernel():
    return pl.kernel(
        _sc_gather_body,
        out_type=jax.ShapeDtypeStruct((_BS, HIDDEN), jnp.float32),
        mesh=plsc.VectorSubcoreMesh(core_axis_name="c", subcore_axis_name="s"),
        scratch_types=[
            pltpu.VMEM((_PER_W,), jnp.int32),
            pltpu.VMEM((2, _CHUNK, HIDDEN), jnp.float32),
            pltpu.SemaphoreType.DMA,
            pltpu.SemaphoreType.DMA,
            pltpu.SemaphoreType.DMA,
            pltpu.SemaphoreType.DMA,
        ],
    )


_CHV = 4
_NV = L // _CHV
_CHT = 16
_NT = S // _CHT


def _layer_norm3(x, gamma, beta):
    mu = jnp.mean(x, axis=-1, keepdims=True)
    xc = x - mu
    var = jnp.mean(xc * xc, axis=-1, keepdims=True)
    return xc * lax.rsqrt(var + EPS) * gamma + beta


def _tc_visual_body(vis_ref, vt_ref, tve_ref, w_ref, b_ref,
                    gamma_ref, beta_ref, out_ref):
    v = vis_ref[...].astype(jnp.bfloat16).reshape(_CHV * B, VIS_DIM)
    proj = jnp.dot(v, w_ref[...], preferred_element_type=jnp.float32)
    proj = proj.reshape(_CHV, B, HIDDEN)
    t = vt_ref[0].astype(jnp.float32)[:, :, None]
    tve0 = tve_ref[0][None, None, :]
    tve_d = (tve_ref[1] - tve_ref[0])[None, None, :]
    ve = proj + b_ref[...][None] + tve0 + t * tve_d
    out_ref[...] = _layer_norm3(ve, gamma_ref[...][None], beta_ref[...][None])


def _tc_visual(vis_t, vt3, tok_type_vis_emb, proj_Wb, proj_b2, gamma2, beta2):
    return pl.pallas_call(
        _tc_visual_body,
        grid=(_NV,),
        in_specs=[
            pl.BlockSpec((_CHV, B, VIS_DIM), lambda g: (g, 0, 0)),
            pl.BlockSpec((1, _CHV, B), lambda g: (g, 0, 0)),
            pl.BlockSpec((2, HIDDEN), lambda g: (0, 0)),
            pl.BlockSpec((VIS_DIM, HIDDEN), lambda g: (0, 0)),
            pl.BlockSpec((1, HIDDEN), lambda g: (0, 0)),
            pl.BlockSpec((1, HIDDEN), lambda g: (0, 0)),
            pl.BlockSpec((1, HIDDEN), lambda g: (0, 0)),
        ],
        out_specs=pl.BlockSpec((_CHV, B, HIDDEN),
                               lambda g: (S // _CHV + g, 0, 0)),
        out_shape=jax.ShapeDtypeStruct((S + L, B, HIDDEN), jnp.float32),
    )(vis_t, vt3, tok_type_vis_emb, proj_Wb, proj_b2, gamma2, beta2)


def _tc_text_body(buf_ref, gw_ref, pos_ref, tt_ref, tte_ref,
                  gamma_ref, beta_ref, out_ref):
    del buf_ref
    t = tt_ref[0].astype(jnp.float32)[:, :, None]
    tte0 = tte_ref[0][None, None, :]
    tte_d = (tte_ref[1] - tte_ref[0])[None, None, :]
    e = gw_ref[...] + pos_ref[0][:, None, :] + tte0 + t * tte_d
    out_ref[...] = _layer_norm3(e, gamma_ref[...][None], beta_ref[...][None])


def _tc_text(buf, gw_t, pos3, tt3, tok_type_emb, gamma2, beta2):
    return pl.pallas_call(
        _tc_text_body,
        grid=(_NT,),
        in_specs=[
            pl.BlockSpec(memory_space=pl.ANY),
            pl.BlockSpec((_CHT, B, HIDDEN), lambda g: (g, 0, 0)),
            pl.BlockSpec((1, _CHT, HIDDEN), lambda g: (g, 0, 0)),
            pl.BlockSpec((1, _CHT, B), lambda g: (g, 0, 0)),
            pl.BlockSpec((2, HIDDEN), lambda g: (0, 0)),
            pl.BlockSpec((1, HIDDEN), lambda g: (0, 0)),
            pl.BlockSpec((1, HIDDEN), lambda g: (0, 0)),
        ],
        out_specs=pl.BlockSpec((_CHT, B, HIDDEN), lambda g: (g, 0, 0)),
        out_shape=jax.ShapeDtypeStruct((S + L, B, HIDDEN), jnp.float32),
        input_output_aliases={0: 0},
    )(buf, gw_t, pos3, tt3, tok_type_emb, gamma2, beta2)


def kernel(input_ids, token_type_ids, visual_embeddings, visual_embeddings_type,
           word_emb, pos_emb, tok_type_emb, tok_type_vis_emb,
           proj_W, proj_b, ln_gamma, ln_beta):
    idx_t = input_ids.astype(jnp.int32).T.reshape(-1)
    gw_t = _sc_gather_kernel()(word_emb, idx_t).reshape(S, B, HIDDEN)
    tt3 = token_type_ids.astype(jnp.int32).T.reshape(_NT, _CHT, B)
    vt3 = visual_embeddings_type.astype(jnp.int32).T.reshape(_NV, _CHV, B)
    vis_t = jnp.transpose(visual_embeddings, (1, 0, 2))
    gamma2 = ln_gamma.reshape(1, HIDDEN)
    beta2 = ln_beta.reshape(1, HIDDEN)
    vbuf = _tc_visual(vis_t, vt3, tok_type_vis_emb,
                      proj_W.astype(jnp.bfloat16), proj_b.reshape(1, HIDDEN),
                      gamma2, beta2)
    out_t = _tc_text(vbuf, gw_t, pos_emb.reshape(_NT, _CHT, HIDDEN), tt3,
                     tok_type_emb, gamma2, beta2)
    return jnp.transpose(out_t, (1, 0, 2))


# speedup vs baseline: 3.2021x; 1.0429x over previous
import functools

import jax
import jax.numpy as jnp
from jax import lax
from jax.experimental import pallas as pl
from jax.experimental.pallas import tpu as pltpu
from jax.experimental.pallas import tpu_sc as plsc

VOCAB = 30522
HIDDEN = 768
B, S, L = 64, 512, 100
VIS_DIM = 2048
EPS = 1e-12


_NC, _NS = 2, 16
_NW = _NC * _NS
_BS = B * S
_PER_W = _BS // _NW
_CHUNK = 64
_NCHUNK = _PER_W // _CHUNK


def _sc_gather_body(table_hbm, idx_hbm, out_hbm, idx_v, buf_v,
                    gsem0, gsem1, ssem0, ssem1):
    wid = lax.axis_index("s") * _NC + lax.axis_index("c")
    base = wid * _PER_W
    pltpu.sync_copy(idx_hbm.at[pl.ds(base, _PER_W)], idx_v)

    gsems = (gsem0, gsem1)
    ssems = (ssem0, ssem1)

    def start_gather(i):
        bslot = i % 2
        return pltpu.async_copy(
            table_hbm.at[idx_v.at[pl.ds(i * _CHUNK, _CHUNK)]],
            buf_v.at[bslot], gsems[bslot])

    def start_scatter(i):
        bslot = i % 2
        return pltpu.async_copy(
            buf_v.at[bslot],
            out_hbm.at[pl.ds(base + i * _CHUNK, _CHUNK)], ssems[bslot])

    gathers = [None] * _NCHUNK
    scatters = [None] * _NCHUNK
    gathers[0] = start_gather(0)
    for i in range(_NCHUNK):
        if i + 1 < _NCHUNK:
            if i >= 1:
                scatters[i - 1].wait()
            gathers[i + 1] = start_gather(i + 1)
        gathers[i].wait()
        scatters[i] = start_scatter(i)
    scatters[_NCHUNK - 2].wait()
    scatters[_NCHUNK - 1].wait()


@functools.cache
def _sc_gather_kernel():
    return pl.kernel(
        _sc_gather_body,
        out_type=jax.ShapeDtypeStruct((_BS, HIDDEN), jnp.float32),
        mesh=plsc.VectorSubcoreMesh(core_axis_name="c", subcore_axis_name="s"),
        scratch_types=[
            pltpu.VMEM((_PER_W,), jnp.int32),
            pltpu.VMEM((2, _CHUNK, HIDDEN), jnp.float32),
            pltpu.SemaphoreType.DMA,
            pltpu.SemaphoreType.DMA,
            pltpu.SemaphoreType.DMA,
            pltpu.SemaphoreType.DMA,
        ],
    )


_CHV = 4
_NV = L // _CHV
_CHT = 32
_NT = S // _CHT


def _layer_norm3(x, gamma, beta):
    mu = jnp.mean(x, axis=-1, keepdims=True)
    xc = x - mu
    var = jnp.mean(xc * xc, axis=-1, keepdims=True)
    return xc * lax.rsqrt(var + EPS) * gamma + beta


def _tc_visual_body(vis_ref, vt_ref, tve_ref, w_ref, b_ref,
                    gamma_ref, beta_ref, out_ref):
    v = vis_ref[...].astype(jnp.bfloat16).reshape(_CHV * B, VIS_DIM)
    proj = jnp.dot(v, w_ref[...], preferred_element_type=jnp.float32)
    proj = proj.reshape(_CHV, B, HIDDEN)
    t = vt_ref[0].astype(jnp.float32)[:, :, None]
    tve0 = tve_ref[0][None, None, :]
    tve_d = (tve_ref[1] - tve_ref[0])[None, None, :]
    ve = proj + b_ref[...][None] + tve0 + t * tve_d
    out_ref[...] = _layer_norm3(ve, gamma_ref[...][None], beta_ref[...][None])


def _tc_visual(vis_t, vt3, tok_type_vis_emb, proj_Wb, proj_b2, gamma2, beta2):
    return pl.pallas_call(
        _tc_visual_body,
        grid=(_NV,),
        in_specs=[
            pl.BlockSpec((_CHV, B, VIS_DIM), lambda g: (g, 0, 0)),
            pl.BlockSpec((1, _CHV, B), lambda g: (g, 0, 0)),
            pl.BlockSpec((2, HIDDEN), lambda g: (0, 0)),
            pl.BlockSpec((VIS_DIM, HIDDEN), lambda g: (0, 0)),
            pl.BlockSpec((1, HIDDEN), lambda g: (0, 0)),
            pl.BlockSpec((1, HIDDEN), lambda g: (0, 0)),
            pl.BlockSpec((1, HIDDEN), lambda g: (0, 0)),
        ],
        out_specs=pl.BlockSpec((_CHV, B, HIDDEN),
                               lambda g: (S // _CHV + g, 0, 0)),
        out_shape=jax.ShapeDtypeStruct((S + L, B, HIDDEN), jnp.float32),
    )(vis_t, vt3, tok_type_vis_emb, proj_Wb, proj_b2, gamma2, beta2)


def _tc_text_body(buf_ref, gw_ref, pos_ref, tt_ref, tte_ref,
                  gamma_ref, beta_ref, out_ref):
    del buf_ref
    t = tt_ref[0].astype(jnp.float32)[:, :, None]
    tte0 = tte_ref[0][None, None, :]
    tte_d = (tte_ref[1] - tte_ref[0])[None, None, :]
    e = gw_ref[...] + pos_ref[0][:, None, :] + tte0 + t * tte_d
    out_ref[...] = _layer_norm3(e, gamma_ref[...][None], beta_ref[...][None])


def _tc_text(buf, gw_t, pos3, tt3, tok_type_emb, gamma2, beta2):
    return pl.pallas_call(
        _tc_text_body,
        grid=(_NT,),
        in_specs=[
            pl.BlockSpec(memory_space=pl.ANY),
            pl.BlockSpec((_CHT, B, HIDDEN), lambda g: (g, 0, 0)),
            pl.BlockSpec((1, _CHT, HIDDEN), lambda g: (g, 0, 0)),
            pl.BlockSpec((1, _CHT, B), lambda g: (g, 0, 0)),
            pl.BlockSpec((2, HIDDEN), lambda g: (0, 0)),
            pl.BlockSpec((1, HIDDEN), lambda g: (0, 0)),
            pl.BlockSpec((1, HIDDEN), lambda g: (0, 0)),
        ],
        out_specs=pl.BlockSpec((_CHT, B, HIDDEN), lambda g: (g, 0, 0)),
        out_shape=jax.ShapeDtypeStruct((S + L, B, HIDDEN), jnp.float32),
        input_output_aliases={0: 0},
    )(buf, gw_t, pos3, tt3, tok_type_emb, gamma2, beta2)


def kernel(input_ids, token_type_ids, visual_embeddings, visual_embeddings_type,
           word_emb, pos_emb, tok_type_emb, tok_type_vis_emb,
           proj_W, proj_b, ln_gamma, ln_beta):
    idx_t = input_ids.astype(jnp.int32).T.reshape(-1)
    gw_t = _sc_gather_kernel()(word_emb, idx_t).reshape(S, B, HIDDEN)
    tt3 = token_type_ids.astype(jnp.int32).T.reshape(_NT, _CHT, B)
    vt3 = visual_embeddings_type.astype(jnp.int32).T.reshape(_NV, _CHV, B)
    vis_t = jnp.transpose(visual_embeddings, (1, 0, 2))
    gamma2 = ln_gamma.reshape(1, HIDDEN)
    beta2 = ln_beta.reshape(1, HIDDEN)
    vbuf = _tc_visual(vis_t, vt3, tok_type_vis_emb,
                      proj_W.astype(jnp.bfloat16), proj_b.reshape(1, HIDDEN),
                      gamma2, beta2)
    out_t = _tc_text(vbuf, gw_t, pos_emb.reshape(_NT, _CHT, HIDDEN), tt3,
                     tok_type_emb, gamma2, beta2)
    return jnp.transpose(out_t, (1, 0, 2))
